# baseline (speedup 1.0000x reference)
import jax
import jax.numpy as jnp
from jax import lax
from jax.experimental import pallas as pl
from jax.experimental.pallas import tpu as pltpu
from jax.experimental.pallas import tpu_sc as plsc

U, S, F, D = 33, 256, 64, 64
L = 16
NJ = F // L
NC, NS = 2, 16
NW = NC * NS
SF = S * F
NROWS = U * D
RPW = NROWS // NW


def _sc_body(mf_hbm, wv_hbm, bv_hbm, featt_hbm, univ_hbm, flag_hbm, out_hbm,
             m_v, featt_v, univ_v, flag_v, bv_v, wv_v, bu_v, dfl_v, baset_v,
             obuf0, obuf1, sem0, sem1):
    w = lax.axis_index("s") * NC + lax.axis_index("c")

    pltpu.sync_copy(featt_hbm, featt_v)
    pltpu.sync_copy(univ_hbm, univ_v)
    pltpu.sync_copy(flag_hbm, flag_v)
    pltpu.sync_copy(bv_hbm, bv_v)
    pltpu.sync_copy(wv_hbm, wv_v)

    zero16 = jnp.zeros((L,), jnp.float32)
    iota16 = lax.iota(jnp.int32, L)

    for j in range(NJ):
        dsj = pl.ds(L * j, L)
        dfl_v[dsj] = flag_v[pl.ds(D + L * j, L)] - flag_v[dsj]

    def load_universe(u):
        pltpu.sync_copy(mf_hbm.at[pl.ds(u * SF, SF)], m_v)
        upred = jnp.full((L,), u > 0)
        for j in range(NJ):
            dsj = pl.ds(L * j, L)
            uv = jnp.where(upred, univ_v[pl.ds(D + L * j, L)], univ_v[dsj])
            bu_v[dsj] = bv_v[dsj] + uv + flag_v[dsj]

        def dbody(d, c):
            bub = plsc.load_gather(bu_v, [jnp.full((L,), d, jnp.int32)])
            dfb = plsc.load_gather(dfl_v, [jnp.full((L,), d, jnp.int32)])
            for j in range(NJ):
                fmask = (iota16 + (L * j)) == (u - 1)
                baset_v[pl.ds(d * F + L * j, L)] = (
                    featt_v[pl.ds(d * F + L * j, L)] + bub
                    + jnp.where(fmask, dfb, zero16))
            return c

        lax.fori_loop(0, D, dbody, 0)

    def fill_row(d, buf):
        wvb = plsc.load_gather(wv_v, [jnp.full((L,), d, jnp.int32)])
        bt = [baset_v[pl.ds(d * F + L * j, L)] for j in range(NJ)]

        def sbody(s, c):
            for j in range(NJ):
                off = s * F + L * j
                buf[pl.ds(off, L)] = m_v[pl.ds(off, L)] * wvb + bt[j]
            return c

        lax.fori_loop(0, S, sbody, 0, unroll=2)

    def do_row(ri, prev_u, buf, sem, wait_first):
        r = w * RPW + ri
        u = r // D
        d = r % D

        @pl.when(u != prev_u)
        def _():
            load_universe(u)

        if wait_first:
            r2 = r - 2
            pltpu.make_async_copy(
                buf, out_hbm.at[pl.ds(r2 * SF, SF)], sem).wait()
        fill_row(d, buf)
        pltpu.async_copy(buf, out_hbm.at[pl.ds(r * SF, SF)], sem)
        return u

    prev_u = do_row(0, jnp.int32(-1), obuf0, sem0, False)
    prev_u = do_row(1, prev_u, obuf1, sem1, False)

    def pair_body(p, pu):
        pu = do_row(2 * p, pu, obuf0, sem0, True)
        pu = do_row(2 * p + 1, pu, obuf1, sem1, True)
        return pu

    lax.fori_loop(1, RPW // 2, pair_body, prev_u)

    r_a = w * RPW + RPW - 2
    pltpu.make_async_copy(
        obuf0, out_hbm.at[pl.ds(r_a * SF, SF)], sem0).wait()
    r_b = w * RPW + RPW - 1
    pltpu.make_async_copy(
        obuf1, out_hbm.at[pl.ds(r_b * SF, SF)], sem1).wait()


@jax.jit
def _sc_embed(mf, wv, bv, featt, univ, flag):
    mesh = plsc.VectorSubcoreMesh(
        core_axis_name="c", subcore_axis_name="s",
        num_cores=NC, num_subcores=NS)
    run = pl.kernel(
        _sc_body,
        out_type=jax.ShapeDtypeStruct((NROWS * SF,), jnp.float32),
        mesh=mesh,
        compiler_params=pltpu.CompilerParams(needs_layout_passes=False),
        scratch_types=[
            pltpu.VMEM((SF,), jnp.float32),
            pltpu.VMEM((D * F,), jnp.float32),
            pltpu.VMEM((2 * D,), jnp.float32),
            pltpu.VMEM((2 * D,), jnp.float32),
            pltpu.VMEM((D,), jnp.float32),
            pltpu.VMEM((D,), jnp.float32),
            pltpu.VMEM((D,), jnp.float32),
            pltpu.VMEM((D,), jnp.float32),
            pltpu.VMEM((D * F,), jnp.float32),
            pltpu.VMEM((SF,), jnp.float32),
            pltpu.VMEM((SF,), jnp.float32),
            pltpu.SemaphoreType.DMA,
            pltpu.SemaphoreType.DMA,
        ],
    )
    return run(mf, wv, bv, featt, univ, flag)


def kernel(m_data, Wv, bv, feat_table, univ_table, flag_table):
    mf = m_data.reshape(U * S * F)
    wv = Wv.reshape(D)
    featt = feat_table.T.reshape(D * F)
    out = _sc_embed(mf, wv, bv, featt,
                    univ_table.reshape(2 * D), flag_table.reshape(2 * D))
    return out.reshape(U, D, SF).transpose(0, 2, 1)


# speedup vs baseline: 2.5474x; 2.5474x over previous
import jax
import jax.numpy as jnp
from jax import lax
from jax.experimental import pallas as pl
from jax.experimental.pallas import tpu as pltpu
from jax.experimental.pallas import tpu_sc as plsc

U, S, F, D = 33, 256, 64, 64
L = 16
NJ = F // L
NC, NS = 2, 16
NW = NC * NS
SF = S * F
NROWS = U * D
RPW = NROWS // NW


def _sc_body(mf_hbm, wv_hbm, bv_hbm, featt_hbm, univ_hbm, flag_hbm, out_hbm,
             m_v, featt_v, univ_v, flag_v, bv_v, wv_v, bu_v, dfl_v, baset_v,
             obuf0, obuf1, sem0, sem1):
    w = lax.axis_index("s") * NC + lax.axis_index("c")

    pltpu.sync_copy(featt_hbm, featt_v)
    pltpu.sync_copy(univ_hbm, univ_v)
    pltpu.sync_copy(flag_hbm, flag_v)
    pltpu.sync_copy(bv_hbm, bv_v)
    pltpu.sync_copy(wv_hbm, wv_v)

    zero16 = jnp.zeros((L,), jnp.float32)
    iota16 = lax.iota(jnp.int32, L)

    for j in range(NJ):
        dsj = pl.ds(L * j, L)
        dfl_v[dsj] = flag_v[pl.ds(D + L * j, L)] - flag_v[dsj]

    def load_universe(u):
        pltpu.sync_copy(mf_hbm.at[pl.ds(u * SF, SF)], m_v)
        upred = jnp.full((L,), u > 0)
        for j in range(NJ):
            dsj = pl.ds(L * j, L)
            uv = jnp.where(upred, univ_v[pl.ds(D + L * j, L)], univ_v[dsj])
            bu_v[dsj] = bv_v[dsj] + uv + flag_v[dsj]

        @plsc.parallel_loop(0, D, step=1, unroll=2)
        def dbody(d):
            bub = plsc.load_gather(bu_v, [jnp.full((L,), d, jnp.int32)])
            dfb = plsc.load_gather(dfl_v, [jnp.full((L,), d, jnp.int32)])
            for j in range(NJ):
                fmask = (iota16 + (L * j)) == (u - 1)
                baset_v[pl.ds(d * F + L * j, L)] = (
                    featt_v[pl.ds(d * F + L * j, L)] + bub
                    + jnp.where(fmask, dfb, zero16))

    def fill_row(d, buf):
        wvb = plsc.load_gather(wv_v, [jnp.full((L,), d, jnp.int32)])
        bt = [baset_v[pl.ds(d * F + L * j, L)] for j in range(NJ)]

        @plsc.parallel_loop(0, S, step=1, unroll=4)
        def sbody(s):
            for j in range(NJ):
                off = s * F + L * j
                buf[pl.ds(off, L)] = m_v[pl.ds(off, L)] * wvb + bt[j]

    def do_row(ri, prev_u, buf, sem, wait_first):
        r = w * RPW + ri
        u = r // D
        d = r % D

        @pl.when(u != prev_u)
        def _():
            load_universe(u)

        if wait_first:
            r2 = r - 2
            pltpu.make_async_copy(
                buf, out_hbm.at[pl.ds(r2 * SF, SF)], sem).wait()
        fill_row(d, buf)
        pltpu.async_copy(buf, out_hbm.at[pl.ds(r * SF, SF)], sem)
        return u

    prev_u = do_row(0, jnp.int32(-1), obuf0, sem0, False)
    prev_u = do_row(1, prev_u, obuf1, sem1, False)

    def pair_body(p, pu):
        pu = do_row(2 * p, pu, obuf0, sem0, True)
        pu = do_row(2 * p + 1, pu, obuf1, sem1, True)
        return pu

    lax.fori_loop(1, RPW // 2, pair_body, prev_u)

    r_a = w * RPW + RPW - 2
    pltpu.make_async_copy(
        obuf0, out_hbm.at[pl.ds(r_a * SF, SF)], sem0).wait()
    r_b = w * RPW + RPW - 1
    pltpu.make_async_copy(
        obuf1, out_hbm.at[pl.ds(r_b * SF, SF)], sem1).wait()


@jax.jit
def _sc_embed(mf, wv, bv, featt, univ, flag):
    mesh = plsc.VectorSubcoreMesh(
        core_axis_name="c", subcore_axis_name="s",
        num_cores=NC, num_subcores=NS)
    run = pl.kernel(
        _sc_body,
        out_type=jax.ShapeDtypeStruct((NROWS * SF,), jnp.float32),
        mesh=mesh,
        compiler_params=pltpu.CompilerParams(needs_layout_passes=False),
        scratch_types=[
            pltpu.VMEM((SF,), jnp.float32),
            pltpu.VMEM((D * F,), jnp.float32),
            pltpu.VMEM((2 * D,), jnp.float32),
            pltpu.VMEM((2 * D,), jnp.float32),
            pltpu.VMEM((D,), jnp.float32),
            pltpu.VMEM((D,), jnp.float32),
            pltpu.VMEM((D,), jnp.float32),
            pltpu.VMEM((D,), jnp.float32),
            pltpu.VMEM((D * F,), jnp.float32),
            pltpu.VMEM((SF,), jnp.float32),
            pltpu.VMEM((SF,), jnp.float32),
            pltpu.SemaphoreType.DMA,
            pltpu.SemaphoreType.DMA,
        ],
    )
    return run(mf, wv, bv, featt, univ, flag)


def kernel(m_data, Wv, bv, feat_table, univ_table, flag_table):
    mf = m_data.reshape(U * S * F)
    wv = Wv.reshape(D)
    featt = feat_table.T.reshape(D * F)
    out = _sc_embed(mf, wv, bv, featt,
                    univ_table.reshape(2 * D), flag_table.reshape(2 * D))
    return out.reshape(U, D, SF).transpose(0, 2, 1)


# speedup vs baseline: 7.1304x; 2.7991x over previous
import jax
import jax.numpy as jnp
from jax import lax
from jax.experimental import pallas as pl
from jax.experimental.pallas import tpu as pltpu
from jax.experimental.pallas import tpu_sc as plsc

U, S, F, D = 33, 256, 64, 64
L = 16
NJ = F // L
NC, NS = 2, 16
NW = NC * NS
SF = S * F
TD, TS = 8, 128
NTR = D // TD
NQ = 4
QSF = SF // NQ
QTC = QSF // TS
UNIT = TD * QSF
TASKS = U * NTR * NQ
TPW = TASKS // NW
UPT = NTR * NQ


def _sc_body(mf_hbm, wv_hbm, bv_hbm, featt_hbm, univ_hbm, flag_hbm, out_hbm,
             m_v, featt_v, univ_v, flag_v, bv_v, wv_v, bu_v, dfl_v, baset_v,
             obuf0, obuf1, sem0, sem1):
    w = lax.axis_index("s") * NC + lax.axis_index("c")

    pltpu.sync_copy(featt_hbm, featt_v)
    pltpu.sync_copy(univ_hbm, univ_v)
    pltpu.sync_copy(flag_hbm, flag_v)
    pltpu.sync_copy(bv_hbm, bv_v)
    pltpu.sync_copy(wv_hbm, wv_v)

    zero16 = jnp.zeros((L,), jnp.float32)
    iota16 = lax.iota(jnp.int32, L)

    for j in range(NJ):
        dsj = pl.ds(L * j, L)
        dfl_v[dsj] = flag_v[pl.ds(D + L * j, L)] - flag_v[dsj]

    def load_universe(u):
        pltpu.sync_copy(mf_hbm.at[pl.ds(u * SF, SF)], m_v)
        upred = jnp.full((L,), u > 0)
        for j in range(NJ):
            dsj = pl.ds(L * j, L)
            uv = jnp.where(upred, univ_v[pl.ds(D + L * j, L)], univ_v[dsj])
            bu_v[dsj] = bv_v[dsj] + uv + flag_v[dsj]

        @plsc.parallel_loop(0, D, step=1, unroll=2)
        def dbody(d):
            bub = plsc.load_gather(bu_v, [jnp.full((L,), d, jnp.int32)])
            dfb = plsc.load_gather(dfl_v, [jnp.full((L,), d, jnp.int32)])
            for j in range(NJ):
                fmask = (iota16 + (L * j)) == (u - 1)
                baset_v[pl.ds(d * F + L * j, L)] = (
                    featt_v[pl.ds(d * F + L * j, L)] + bub
                    + jnp.where(fmask, dfb, zero16))

    def fill_unit(tr, q, buf):
        def ddbody(dd, c):
            d = tr * TD + dd
            wvb = plsc.load_gather(wv_v, [jnp.full((L,), d, jnp.int32)])
            bt = [baset_v[pl.ds(d * F + L * j, L)] for j in range(NJ)]

            @plsc.parallel_loop(0, QSF // F, step=1, unroll=4)
            def ibody(i4):
                for jj in range(NJ):
                    moff = q * QSF + i4 * F + L * jj
                    pos = i4 * NJ + jj
                    boff = ((pos // 8) * (TD * TS) + dd * TS
                            + (pos % 8) * L)
                    buf[pl.ds(boff, L)] = (
                        m_v[pl.ds(moff, L)] * wvb + bt[jj])
            return c

        lax.fori_loop(0, TD, ddbody, 0)

    def do_unit(ti, prev_u, buf, sem, wait_first):
        t = w * TPW + ti
        u = t // UPT
        rem = t % UPT
        tr = rem // NQ
        q = rem % NQ

        @pl.when(u != prev_u)
        def _():
            load_universe(u)

        if wait_first:
            t2 = t - 2
            pltpu.make_async_copy(
                buf, out_hbm.at[pl.ds(t2 * UNIT, UNIT)], sem).wait()
        fill_unit(tr, q, buf)
        pltpu.async_copy(buf, out_hbm.at[pl.ds(t * UNIT, UNIT)], sem)
        return u

    prev_u = do_unit(0, jnp.int32(-1), obuf0, sem0, False)
    prev_u = do_unit(1, prev_u, obuf1, sem1, False)

    def pair_body(p, pu):
        pu = do_unit(2 * p, pu, obuf0, sem0, True)
        pu = do_unit(2 * p + 1, pu, obuf1, sem1, True)
        return pu

    lax.fori_loop(1, TPW // 2, pair_body, prev_u)

    t_a = w * TPW + TPW - 2
    pltpu.make_async_copy(
        obuf0, out_hbm.at[pl.ds(t_a * UNIT, UNIT)], sem0).wait()
    t_b = w * TPW + TPW - 1
    pltpu.make_async_copy(
        obuf1, out_hbm.at[pl.ds(t_b * UNIT, UNIT)], sem1).wait()


@jax.jit
def _sc_embed(mf, wv, bv, featt, univ, flag):
    mesh = plsc.VectorSubcoreMesh(
        core_axis_name="c", subcore_axis_name="s",
        num_cores=NC, num_subcores=NS)
    run = pl.kernel(
        _sc_body,
        out_type=jax.ShapeDtypeStruct((TASKS * UNIT,), jnp.float32),
        mesh=mesh,
        compiler_params=pltpu.CompilerParams(needs_layout_passes=False),
        scratch_types=[
            pltpu.VMEM((SF,), jnp.float32),
            pltpu.VMEM((D * F,), jnp.float32),
            pltpu.VMEM((2 * D,), jnp.float32),
            pltpu.VMEM((2 * D,), jnp.float32),
            pltpu.VMEM((D,), jnp.float32),
            pltpu.VMEM((D,), jnp.float32),
            pltpu.VMEM((D,), jnp.float32),
            pltpu.VMEM((D,), jnp.float32),
            pltpu.VMEM((D * F,), jnp.float32),
            pltpu.VMEM((UNIT,), jnp.float32),
            pltpu.VMEM((UNIT,), jnp.float32),
            pltpu.SemaphoreType.DMA,
            pltpu.SemaphoreType.DMA,
        ],
    )
    return run(mf, wv, bv, featt, univ, flag)


def kernel(m_data, Wv, bv, feat_table, univ_table, flag_table):
    mf = m_data.reshape(U * S * F)
    wv = Wv.reshape(D)
    featt = feat_table.T.reshape(D * F)
    out = _sc_embed(mf, wv, bv, featt,
                    univ_table.reshape(2 * D), flag_table.reshape(2 * D))
    out = (out.reshape(U, NTR, SF // TS, TD, TS)
           .transpose(0, 1, 3, 2, 4)
           .reshape(U, D, SF)
           .transpose(0, 2, 1))
    return out
